# Initial kernel scaffold; baseline (speedup 1.0000x reference)
import jax
import jax.numpy as jnp
from jax.experimental import pallas as pl


def kernel(input_pos, output_pos, output_neg, W_in, W_out):
    raise NotImplementedError("write your pallas kernel here")


# speedup vs baseline: 4.1040x; 4.1040x over previous
import functools

import jax
import jax.numpy as jnp
from jax import lax
from jax.experimental import pallas as pl
from jax.experimental.pallas import tpu as pltpu
from jax.experimental.pallas import tpu_sc as plsc

B = 16384
D = 64
K = 20
NC = 2
NS = 16
NW = NC * NS
CHUNK = 128

CA = B // NW // CHUNK
CP = B // NW // CHUNK
CN = B * K // NW // CHUNK


def _sc_gather(ip2d, op2d, on2d, W_in, W_out):
    mesh = plsc.VectorSubcoreMesh(core_axis_name="c", subcore_axis_name="s")

    @functools.partial(
        pl.kernel,
        out_type=(
            jax.ShapeDtypeStruct((B, D), jnp.float32),
            jax.ShapeDtypeStruct((B, D), jnp.float32),
            jax.ShapeDtypeStruct((B * K, D), jnp.float32),
        ),
        mesh=mesh,
        scratch_types=[
            pltpu.VMEM((CA, CHUNK), jnp.int32),
            pltpu.VMEM((CP, CHUNK), jnp.int32),
            pltpu.VMEM((CN, CHUNK), jnp.int32),
            pltpu.VMEM((CHUNK, D), jnp.float32),
            pltpu.SemaphoreType.DMA,
        ],
        compiler_params=pltpu.CompilerParams(use_tc_tiling_on_sc=False),
    )
    def k(ip_hbm, op_hbm, on_hbm, win_hbm, wout_hbm,
          out_in, out_pos, out_neg, ia_v, ip_v, in_v, rows_v, sem):
        wid = lax.axis_index("s") * NC + lax.axis_index("c")
        pltpu.sync_copy(ip_hbm.at[pl.ds(wid * CA, CA)], ia_v)
        pltpu.sync_copy(op_hbm.at[pl.ds(wid * CP, CP)], ip_v)
        pltpu.sync_copy(on_hbm.at[pl.ds(wid * CN, CN)], in_v)

        for c in range(CA):
            pltpu.async_copy(win_hbm.at[ia_v.at[c]], rows_v, sem).wait()
            pltpu.sync_copy(rows_v, out_in.at[pl.ds((wid * CA + c) * CHUNK, CHUNK)])
        for c in range(CP):
            pltpu.async_copy(wout_hbm.at[ip_v.at[c]], rows_v, sem).wait()
            pltpu.sync_copy(rows_v, out_pos.at[pl.ds((wid * CP + c) * CHUNK, CHUNK)])

        def body(c, _):
            pltpu.async_copy(wout_hbm.at[in_v.at[c]], rows_v, sem).wait()
            pltpu.sync_copy(rows_v, out_neg.at[pl.ds((wid * CN + c) * CHUNK, CHUNK)])
            return _
        lax.fori_loop(0, CN, body, None)

    return k(ip2d, op2d, on2d, W_in, W_out)


_BB = 1024


def _log_sigmoid(v):
    return jnp.minimum(v, 0.0) - jnp.log1p(jnp.exp(-jnp.abs(v)))


def _loss_body(in_ref, pos_ref, neg_ref, out_ref):
    i = pl.program_id(0)
    x = in_ref[...]
    p = pos_ref[...]
    n = neg_ref[...].reshape(_BB, K, D)
    pos_sim = jnp.clip(jnp.sum(x * p, axis=1), -10.0, 10.0)
    neg_sim = jnp.clip(jnp.sum(n * x[:, None, :], axis=2), -10.0, 10.0)
    tot = jnp.sum(_log_sigmoid(pos_sim)) + jnp.sum(_log_sigmoid(-neg_sim))

    @pl.when(i == 0)
    def _():
        out_ref[...] = jnp.zeros_like(out_ref)
    out_ref[...] += tot[None, None]


def _tc_loss(in_rows, pos_rows, neg_rows):
    grid = (B // _BB,)
    res = pl.pallas_call(
        _loss_body,
        grid=grid,
        in_specs=[
            pl.BlockSpec((_BB, D), lambda i: (i, 0)),
            pl.BlockSpec((_BB, D), lambda i: (i, 0)),
            pl.BlockSpec((_BB * K, D), lambda i: (i, 0)),
        ],
        out_specs=pl.BlockSpec((1, 1), lambda i: (0, 0)),
        out_shape=jax.ShapeDtypeStruct((1, 1), jnp.float32),
    )(in_rows, pos_rows, neg_rows)
    return res


def kernel(input_pos, output_pos, output_neg, W_in, W_out):
    ip2d = input_pos.reshape(-1, CHUNK)
    op2d = output_pos.reshape(-1, CHUNK)
    on2d = output_neg.reshape(-1, CHUNK)
    in_rows, pos_rows, neg_rows = _sc_gather(ip2d, op2d, on2d, W_in, W_out)
    total = _tc_loss(in_rows, pos_rows, neg_rows)
    return -total[0, 0] / B
